# baseline (speedup 1.0000x reference)
import functools

import jax
import jax.numpy as jnp
from jax import lax
from jax.experimental import pallas as pl
from jax.experimental.pallas import tpu as pltpu
from jax.experimental.pallas import tpu_sc as plsc


def _concat_tables(old2, new2, n_old_blk, n_new_blk, blk):
    def body(old_ref, new_ref, out_ref):
        i = pl.program_id(0)

        @pl.when(i < n_old_blk)
        def _():
            out_ref[...] = old_ref[...]

        @pl.when(i >= n_old_blk)
        def _():
            out_ref[...] = new_ref[...]

    total = n_old_blk + n_new_blk
    return pl.pallas_call(
        body,
        grid=(total,),
        in_specs=[
            pl.BlockSpec((blk, 128), lambda i: (jnp.minimum(i, n_old_blk - 1), 0)),
            pl.BlockSpec((blk, 128), lambda i: (jnp.maximum(i - n_old_blk, 0), 0)),
        ],
        out_specs=pl.BlockSpec((blk, 128), lambda i: (i, 0)),
        out_shape=jax.ShapeDtypeStruct((total * blk, 128), jnp.float32),
    )(old2, new2)


def kernel(input_ids, old_weight, new_weight):
    old_vocab, d = old_weight.shape
    new_vocab = new_weight.shape[0]
    batch, hist = input_ids.shape

    ids_t = input_ids.astype(jnp.int32).T

    packf = 128 // d
    blk = 1000
    n_old_blk = old_vocab // packf // blk
    n_new_blk = new_vocab // packf // blk
    combined = _concat_tables(
        old_weight.reshape(old_vocab // packf, 128),
        new_weight.reshape(new_vocab // packf, 128),
        n_old_blk,
        n_new_blk,
        blk,
    ).reshape(old_vocab + new_vocab, d)

    info = plsc.get_sparse_core_info()
    nc, ns, nl = info.num_cores, info.num_subcores, info.num_lanes
    nw = nc * ns
    bw = batch // nw
    hb_n = hist // 8

    mesh = plsc.VectorSubcoreMesh(core_axis_name="c", subcore_axis_name="s")

    dt_n = d // 8
    @functools.partial(
        pl.kernel,
        mesh=mesh,
        compiler_params=pltpu.CompilerParams(
            use_tc_tiling_on_sc=False, needs_layout_passes=False
        ),
        out_type=jax.ShapeDtypeStruct((hist, dt_n, nw, 8 * bw), jnp.float32),
        scratch_types=[
            pltpu.VMEM((8, bw), jnp.int32),
            pltpu.VMEM((bw, d), jnp.float32),
            pltpu.VMEM((d * bw,), jnp.float32),
            pltpu.SemaphoreType.DMA,
            pltpu.SemaphoreType.DMA,
        ],
    )
    def gather_k(tbl_hbm, ids_hbm, out_hbm, ids_blk, rows_v, slab_v, sem, sem2):
        wid = lax.axis_index("s") * nc + lax.axis_index("c")
        bt = wid * bw
        lane = lax.broadcasted_iota(jnp.int32, (nl,), 0)
        lpart = ((lane >> 3) << 10) + ((lane & 7) << 7)

        def hb_body(hb, carry):
            pltpu.sync_copy(ids_hbm.at[pl.ds(hb * 8, 8), pl.ds(bt, bw)], ids_blk)
            for hl in range(8):
                pltpu.async_copy(tbl_hbm.at[ids_blk.at[hl]], rows_v, sem).wait()

                def r_body(r, c2):
                    for s in range(d // nl):
                        v = rows_v[r, pl.ds(s * nl, nl)]
                        plsc.store_scatter(
                            slab_v, [lpart + (s * 2 * 8 * bw + r)], v
                        )
                    return c2

                lax.fori_loop(0, bw, r_body, 0)
                copies = [
                    pltpu.async_copy(
                        slab_v.at[pl.ds(dt * 8 * bw, 8 * bw)],
                        out_hbm.at[hb * 8 + hl, dt, wid],
                        sem2,
                    )
                    for dt in range(dt_n)
                ]
                for cp in copies:
                    cp.wait()
            return carry

        lax.fori_loop(0, hb_n, hb_body, 0)

    out5 = gather_k(combined, ids_t)
    out6 = out5.reshape(hist, dt_n, nw, 8, bw)
    return jnp.transpose(out6, (2, 4, 0, 1, 3)).reshape(batch, hist, d)


# speedup vs baseline: 1.2061x; 1.2061x over previous
import functools

import jax
import jax.numpy as jnp
from jax import lax
from jax.experimental import pallas as pl
from jax.experimental.pallas import tpu as pltpu
from jax.experimental.pallas import tpu_sc as plsc


def _concat_tables(old2, new2, n_old_blk, n_new_blk, blk):
    def body(old_ref, new_ref, out_ref):
        i = pl.program_id(0)

        @pl.when(i < n_old_blk)
        def _():
            out_ref[...] = old_ref[...]

        @pl.when(i >= n_old_blk)
        def _():
            out_ref[...] = new_ref[...]

    total = n_old_blk + n_new_blk
    return pl.pallas_call(
        body,
        grid=(total,),
        in_specs=[
            pl.BlockSpec((blk, 128), lambda i: (jnp.minimum(i, n_old_blk - 1), 0)),
            pl.BlockSpec((blk, 128), lambda i: (jnp.maximum(i - n_old_blk, 0), 0)),
        ],
        out_specs=pl.BlockSpec((blk, 128), lambda i: (i, 0)),
        out_shape=jax.ShapeDtypeStruct((total * blk, 128), jnp.float32),
    )(old2, new2)


def kernel(input_ids, old_weight, new_weight):
    old_vocab, d = old_weight.shape
    new_vocab = new_weight.shape[0]
    batch, hist = input_ids.shape

    ids_t = input_ids.astype(jnp.int32).T

    packf = 128 // d
    blk = 1000
    n_old_blk = old_vocab // packf // blk
    n_new_blk = new_vocab // packf // blk
    combined = _concat_tables(
        old_weight.reshape(old_vocab // packf, 128),
        new_weight.reshape(new_vocab // packf, 128),
        n_old_blk,
        n_new_blk,
        blk,
    ).reshape(old_vocab + new_vocab, d)

    info = plsc.get_sparse_core_info()
    nc, ns, nl = info.num_cores, info.num_subcores, info.num_lanes
    nw = nc * ns
    bw = batch // nw
    hb_n = hist // 8

    mesh = plsc.VectorSubcoreMesh(core_axis_name="c", subcore_axis_name="s")

    dt_n = d // 8
    @functools.partial(
        pl.kernel,
        mesh=mesh,
        compiler_params=pltpu.CompilerParams(
            use_tc_tiling_on_sc=False, needs_layout_passes=False
        ),
        out_type=jax.ShapeDtypeStruct((hist, dt_n, nw, 8 * bw), jnp.float32),
        scratch_types=[
            pltpu.VMEM((8, bw), jnp.int32),
            pltpu.VMEM((8, bw, d), jnp.float32),
            pltpu.VMEM((d * bw,), jnp.float32),
            pltpu.VMEM((d * bw,), jnp.float32),
            pltpu.SemaphoreType.DMA,
            pltpu.SemaphoreType.DMA,
        ],
    )
    def gather_k(
        tbl_hbm, ids_hbm, out_hbm, ids_blk, rows8, slab_a, slab_b, sem, sem2
    ):
        wid = lax.axis_index("s") * nc + lax.axis_index("c")
        bt = wid * bw
        lane = lax.broadcasted_iota(jnp.int32, (nl,), 0)
        lpart = ((lane >> 3) << 10) + ((lane & 7) << 7)
        slabs = (slab_a, slab_b)

        def slab_out_copies(slab_v, h, construct_only=False):
            mk = pltpu.make_async_copy if construct_only else pltpu.async_copy
            return [
                mk(
                    slab_v.at[pl.ds(dt * 8 * bw, 8 * bw)],
                    out_hbm.at[h, dt, wid],
                    sem2,
                )
                for dt in range(dt_n)
            ]

        def hb_body(hb, carry):
            pltpu.sync_copy(ids_hbm.at[pl.ds(hb * 8, 8), pl.ds(bt, bw)], ids_blk)
            gathers = [
                pltpu.async_copy(tbl_hbm.at[ids_blk.at[hl]], rows8.at[hl], sem)
                for hl in range(8)
            ]
            for hl in range(8):
                slab_v = slabs[hl % 2]
                gathers[hl].wait()
                drains = slab_out_copies(
                    slab_v, hb * 8 + hl - 2, construct_only=True
                )
                if hl >= 2:
                    for cp in drains:
                        cp.wait()
                else:

                    @pl.when(hb > 0)
                    def _():
                        for cp in drains:
                            cp.wait()

                def r_body(r, c2):
                    idx0 = lpart + r
                    for s in range(d // nl):
                        v = rows8[hl, r, pl.ds(s * nl, nl)]
                        plsc.store_scatter(slab_v, [idx0 + s * 2 * 8 * bw], v)
                    return c2

                lax.fori_loop(0, bw, r_body, 0, unroll=8)
                slab_out_copies(slab_v, hb * 8 + hl)
            return carry

        lax.fori_loop(0, hb_n, hb_body, 0)
        for hl in (6, 7):
            for cp in slab_out_copies(slabs[hl % 2], hb_n * 8 + hl - 8):
                cp.wait()

    out5 = gather_k(combined, ids_t)
    out6 = out5.reshape(hist, dt_n, nw, 8, bw)
    return jnp.transpose(out6, (2, 4, 0, 1, 3)).reshape(batch, hist, d)


# speedup vs baseline: 1.5929x; 1.3207x over previous
import functools

import jax
import jax.numpy as jnp
from jax import lax
from jax.experimental import pallas as pl
from jax.experimental.pallas import tpu as pltpu
from jax.experimental.pallas import tpu_sc as plsc


def _concat_tables(old2, new2, n_old_blk, n_new_blk, blk):
    def body(old_ref, new_ref, out_ref):
        i = pl.program_id(0)

        @pl.when(i < n_old_blk)
        def _():
            out_ref[...] = old_ref[...]

        @pl.when(i >= n_old_blk)
        def _():
            out_ref[...] = new_ref[...]

    total = n_old_blk + n_new_blk
    return pl.pallas_call(
        body,
        grid=(total,),
        in_specs=[
            pl.BlockSpec((blk, 128), lambda i: (jnp.minimum(i, n_old_blk - 1), 0)),
            pl.BlockSpec((blk, 128), lambda i: (jnp.maximum(i - n_old_blk, 0), 0)),
        ],
        out_specs=pl.BlockSpec((blk, 128), lambda i: (i, 0)),
        out_shape=jax.ShapeDtypeStruct((total * blk, 128), jnp.float32),
    )(old2, new2)


def kernel(input_ids, old_weight, new_weight):
    old_vocab, d = old_weight.shape
    new_vocab = new_weight.shape[0]
    batch, hist = input_ids.shape

    ids_t = input_ids.astype(jnp.int32).T

    packf = 128 // d
    blk = 1000
    n_old_blk = old_vocab // packf // blk
    n_new_blk = new_vocab // packf // blk
    combined = _concat_tables(
        old_weight.reshape(old_vocab // packf, 128),
        new_weight.reshape(new_vocab // packf, 128),
        n_old_blk,
        n_new_blk,
        blk,
    ).reshape(old_vocab + new_vocab, d)

    info = plsc.get_sparse_core_info()
    nc, ns, nl = info.num_cores, info.num_subcores, info.num_lanes
    nw = nc * ns
    bw = batch // nw
    hb_n = hist // 8

    mesh = plsc.VectorSubcoreMesh(core_axis_name="c", subcore_axis_name="s")

    dt_n = d // 8
    @functools.partial(
        pl.kernel,
        mesh=mesh,
        compiler_params=pltpu.CompilerParams(
            use_tc_tiling_on_sc=False, needs_layout_passes=False
        ),
        out_type=jax.ShapeDtypeStruct((hist, dt_n, nw, 8 * bw), jnp.float32),
        scratch_types=[
            pltpu.VMEM((8, bw), jnp.int32),
            pltpu.VMEM((8, bw, d), jnp.float32),
            pltpu.VMEM((d * bw,), jnp.float32),
            pltpu.VMEM((d * bw,), jnp.float32),
            pltpu.VMEM((nl * 17,), jnp.float32),
            pltpu.SemaphoreType.DMA,
            pltpu.SemaphoreType.DMA,
        ],
    )
    def gather_k(
        tbl_hbm, ids_hbm, out_hbm, ids_blk, rows8, slab_a, slab_b, mini, sem, sem2
    ):
        wid = lax.axis_index("s") * nc + lax.axis_index("c")
        bt = wid * bw
        lane = lax.broadcasted_iota(jnp.int32, (nl,), 0)
        lane17 = lane * 17
        slabs = (slab_a, slab_b)

        def slab_out_copies(slab_v, h, construct_only=False):
            mk = pltpu.make_async_copy if construct_only else pltpu.async_copy
            return [
                mk(
                    slab_v.at[pl.ds(dt * 8 * bw, 8 * bw)],
                    out_hbm.at[h, dt, wid],
                    sem2,
                )
                for dt in range(dt_n)
            ]

        def hb_body(hb, carry):
            pltpu.sync_copy(ids_hbm.at[pl.ds(hb * 8, 8), pl.ds(bt, bw)], ids_blk)
            gathers = [
                pltpu.async_copy(tbl_hbm.at[ids_blk.at[hl]], rows8.at[hl], sem)
                for hl in range(8)
            ]
            for hl in range(8):
                slab_v = slabs[hl % 2]
                gathers[hl].wait()
                drains = slab_out_copies(
                    slab_v, hb * 8 + hl - 2, construct_only=True
                )
                if hl >= 2:
                    for cp in drains:
                        cp.wait()
                else:

                    @pl.when(hb > 0)
                    def _():
                        for cp in drains:
                            cp.wait()

                def r0_body(r0, c2):
                    rb = r0 * nl
                    for s in range(d // nl):
                        for rl in range(nl):
                            v = rows8[hl, rb + rl, pl.ds(s * nl, nl)]
                            plsc.store_scatter(mini, [lane17 + rl], v)
                        for dl in range(nl):
                            w = mini[pl.ds(dl * 17, nl)]
                            base = (2 * s + dl // 8) * (8 * bw) + (dl % 8) * bw
                            slab_v[pl.ds(base + rb, nl)] = w
                    return c2

                lax.fori_loop(0, bw // nl, r0_body, 0)
                slab_out_copies(slab_v, hb * 8 + hl)
            return carry

        lax.fori_loop(0, hb_n, hb_body, 0)
        for hl in (6, 7):
            for cp in slab_out_copies(slabs[hl % 2], hb_n * 8 + hl - 8):
                cp.wait()

    out5 = gather_k(combined, ids_t)
    out6 = out5.reshape(hist, dt_n, nw, 8, bw)
    return jnp.transpose(out6, (2, 4, 0, 1, 3)).reshape(batch, hist, d)
